# Initial kernel scaffold; baseline (speedup 1.0000x reference)
import jax
import jax.numpy as jnp
from jax.experimental import pallas as pl


def kernel(boxes, scores, gt_boxes, gt_classes):
    raise NotImplementedError("write your pallas kernel here")


# speedup vs baseline: 6.6750x; 6.6750x over previous
import functools
import jax
import jax.numpy as jnp
from jax import lax
from jax.experimental import pallas as pl
from jax.experimental.pallas import tpu as pltpu
from jax.experimental.pallas import tpu_sc as plsc

_NUM_CLASSES = 80
_SCORE_THRESH = 0.05
_NMS_THRESH = 0.5
_IOU_FG_THRESH = 0.5
_N = 5000
_NP = 5120
_NGT = 64
_K = 100
_KP = 128
_NS = 16
_CHUNK = _NP // _NS
_NB = _CHUNK // 16
_BIG = 0x7FFFFFFF


def _body(px1, py1, px2, py2, psc, gx1, gy1, gx2, gy2, gcls,
          mi_out, pc_out, ki_out, ks_out, kx1_out, ky1_out, kx2_out,
          ky2_out, kcls_out,
          x1v, y1v, x2v, y2v, scv, gx1v, gy1v, gx2v, gy2v, gclsv,
          miv, pcv, rowv, candv, kx1v, ky1v, kx2v, ky2v, ksv, kiv,
          kclsv, cand_sh, cnt):
    sid = lax.axis_index("s")
    base = sid * _CHUNK
    iota = lax.iota(jnp.int32, 16)
    lane0 = iota == 0

    @pl.when(sid == 0)
    def _():
        old = plsc.fetch_and_add(cnt.at[0], 0, subcore_id=0)
        plsc.fetch_and_add(cnt.at[0], -old, subcore_id=0)

    rowv[...] = jnp.zeros((16,), jnp.float32)
    rowv[...] = rowv[...]
    plsc.fetch_and_add(cnt.at[0], 0, subcore_id=0)
    pltpu.sync_copy(rowv, cand_sh.at[pl.ds(sid * 16, 16)])

    pltpu.sync_copy(px1.at[pl.ds(base, _CHUNK)], x1v)
    pltpu.sync_copy(py1.at[pl.ds(base, _CHUNK)], y1v)
    pltpu.sync_copy(px2.at[pl.ds(base, _CHUNK)], x2v)
    pltpu.sync_copy(py2.at[pl.ds(base, _CHUNK)], y2v)
    pltpu.sync_copy(psc.at[pl.ds(base, _CHUNK)], scv)
    pltpu.sync_copy(gx1, gx1v)
    pltpu.sync_copy(gy1, gy1v)
    pltpu.sync_copy(gx2, gx2v)
    pltpu.sync_copy(gy2, gy2v)
    pltpu.sync_copy(gcls, gclsv)

    for b in range(_NB):
        off = b * 16
        q1 = x1v[pl.ds(off, 16)]
        q2 = y1v[pl.ds(off, 16)]
        q3 = x2v[pl.ds(off, 16)]
        q4 = y2v[pl.ds(off, 16)]
        parea = (q3 - q1) * (q4 - q2)

        def gt_body(j, carry):
            biou, bj = carry
            jv = jnp.full((16,), j, jnp.int32)
            g1 = plsc.load_gather(gx1v, [jv])
            g2 = plsc.load_gather(gy1v, [jv])
            g3 = plsc.load_gather(gx2v, [jv])
            g4 = plsc.load_gather(gy2v, [jv])
            garea = (g3 - g1) * (g4 - g2)
            ww = jnp.maximum(jnp.minimum(g3, q3) - jnp.maximum(g1, q1), 0.0)
            hh = jnp.maximum(jnp.minimum(g4, q4) - jnp.maximum(g2, q2), 0.0)
            inter = ww * hh
            iou = inter / jnp.maximum(garea + parea - inter, 1e-8)
            upd = iou > biou
            return jnp.where(upd, iou, biou), jnp.where(upd, jv, bj)

        biou, bj = lax.fori_loop(
            0, _NGT, gt_body,
            (jnp.full((16,), -1.0, jnp.float32), jnp.zeros((16,), jnp.int32)))
        cls_g = plsc.load_gather(gclsv, [bj])
        pcls = jnp.where(biou >= _IOU_FG_THRESH, cls_g,
                         jnp.full((16,), _NUM_CLASSES, jnp.int32))
        miv[pl.ds(off, 16)] = bj
        pcv[pl.ds(off, 16)] = pcls
        s = scv[pl.ds(off, 16)]
        s = jnp.where(s > _SCORE_THRESH, s, -1.0)
        gidx = jnp.full((16,), base + off, jnp.int32) + iota
        scv[pl.ds(off, 16)] = jnp.where(gidx < _N, s, -2.0)

    pltpu.sync_copy(miv, mi_out.at[pl.ds(base, _CHUNK)])
    pltpu.sync_copy(pcv, pc_out.at[pl.ds(base, _CHUNK)])
    plsc.subcore_barrier()

    def nms_body(t, carry):
        bval = jnp.full((16,), -jnp.inf, jnp.float32)
        bblk = jnp.zeros((16,), jnp.int32)
        for b in range(_NB):
            s = scv[pl.ds(b * 16, 16)]
            upd = s > bval
            bval = jnp.where(upd, s, bval)
            bblk = jnp.where(upd, jnp.full((16,), b, jnp.int32), bblk)
        m = jnp.max(bval)
        gi = jnp.full((16,), base, jnp.int32) + bblk * 16 + iota
        li = jnp.min(jnp.where(bval == m, gi, _BIG))
        locv = jnp.full((16,), li - base, jnp.int32)
        c1 = plsc.load_gather(x1v, [locv])
        c2 = plsc.load_gather(y1v, [locv])
        c3 = plsc.load_gather(x2v, [locv])
        c4 = plsc.load_gather(y2v, [locv])
        c5 = plsc.load_gather(pcv, [locv]).astype(jnp.float32)
        lif = jnp.full((16,), li, jnp.int32).astype(jnp.float32)
        mf = jnp.full((16,), m, jnp.float32)
        tagv = jnp.full((16,), t + 1, jnp.int32).astype(jnp.float32)
        row = jnp.where(iota == 0, mf,
              jnp.where(iota == 1, lif,
              jnp.where(iota == 2, c1,
              jnp.where(iota == 3, c2,
              jnp.where(iota == 4, c3,
              jnp.where(iota == 5, c4,
              jnp.where(iota == 6, c5, tagv)))))))
        rowv[...] = row
        rowv[...] = rowv[...]
        plsc.fetch_and_add(cnt.at[0], 0, subcore_id=0)
        pltpu.sync_copy(rowv, cand_sh.at[pl.ds(sid * 16, 16)])

        def spin(target):
            def spin_cond(c):
                return c < target

            def spin_body(c):
                return plsc.fetch_and_add(cnt.at[0], 0, subcore_id=0)

            lax.while_loop(spin_cond, spin_body, jnp.int32(0))

        plsc.fetch_and_add(cnt.at[0], 1, subcore_id=0)
        spin((2 * t + 1) * _NS)

        pltpu.sync_copy(cand_sh, candv)
        rbase = iota * 16
        for _ in range(6):
            tags = plsc.load_gather(candv, [rbase + 7])
            stale = jnp.logical_not(jnp.all(tags == tagv))

            @pl.when(stale)
            def _():
                plsc.fetch_and_add(cnt.at[0], 0, subcore_id=0)
                plsc.fetch_and_add(cnt.at[0], 0, subcore_id=0)
                pltpu.sync_copy(cand_sh, candv)

        plsc.fetch_and_add(cnt.at[0], 1, subcore_id=0)
        spin((2 * t + 2) * _NS)

        sv = plsc.load_gather(candv, [rbase])
        ivf = plsc.load_gather(candv, [rbase + 1])
        iv = ivf.astype(jnp.int32)
        gm = jnp.max(sv)
        winmask = sv == gm
        gbi = jnp.min(jnp.where(winmask, iv, _BIG))
        wlane = jnp.min(jnp.where(winmask & (iv == gbi), iota,
                                  jnp.full((16,), 99, jnp.int32)))
        wb = jnp.full((16,), wlane * 16, jnp.int32)
        bx1 = plsc.load_gather(candv, [wb + 2])
        by1 = plsc.load_gather(candv, [wb + 3])
        bx2 = plsc.load_gather(candv, [wb + 4])
        by2 = plsc.load_gather(candv, [wb + 5])
        bcl = plsc.load_gather(candv, [wb + 6])
        tv = jnp.full((16,), t, jnp.int32)
        plsc.store_scatter(kiv, [tv], jnp.full((16,), gbi, jnp.int32),
                           mask=lane0)
        plsc.store_scatter(ksv, [tv], jnp.full((16,), gm, jnp.float32),
                           mask=lane0)
        plsc.store_scatter(kx1v, [tv], bx1, mask=lane0)
        plsc.store_scatter(ky1v, [tv], by1, mask=lane0)
        plsc.store_scatter(kx2v, [tv], bx2, mask=lane0)
        plsc.store_scatter(ky2v, [tv], by2, mask=lane0)
        plsc.store_scatter(kclsv, [tv], bcl.astype(jnp.int32), mask=lane0)
        barea = (bx2 - bx1) * (by2 - by1)
        for b in range(_NB):
            off = b * 16
            q1 = x1v[pl.ds(off, 16)]
            q2 = y1v[pl.ds(off, 16)]
            q3 = x2v[pl.ds(off, 16)]
            q4 = y2v[pl.ds(off, 16)]
            ww = jnp.maximum(jnp.minimum(bx2, q3) - jnp.maximum(bx1, q1), 0.0)
            hh = jnp.maximum(jnp.minimum(by2, q4) - jnp.maximum(by1, q2), 0.0)
            inter = ww * hh
            qarea = (q3 - q1) * (q4 - q2)
            iou = inter / jnp.maximum(barea + qarea - inter, 1e-8)
            s = scv[pl.ds(off, 16)]
            scv[pl.ds(off, 16)] = jnp.where(iou > _NMS_THRESH, -1.0, s)
        return carry

    lax.fori_loop(0, _K, nms_body, jnp.int32(0))

    @pl.when(sid == 0)
    def _():
        pltpu.sync_copy(kiv, ki_out)
        pltpu.sync_copy(ksv, ks_out)
        pltpu.sync_copy(kx1v, kx1_out)
        pltpu.sync_copy(ky1v, ky1_out)
        pltpu.sync_copy(kx2v, kx2_out)
        pltpu.sync_copy(ky2v, ky2_out)
        pltpu.sync_copy(kclsv, kcls_out)


_mesh = plsc.VectorSubcoreMesh(core_axis_name="c", subcore_axis_name="s",
                               num_cores=1)

_f32 = jnp.float32
_i32 = jnp.int32

_roiheads_sc = pl.kernel(
    _body,
    out_type=(
        jax.ShapeDtypeStruct((_NP,), _i32),
        jax.ShapeDtypeStruct((_NP,), _i32),
        jax.ShapeDtypeStruct((_KP,), _i32),
        jax.ShapeDtypeStruct((_KP,), _f32),
        jax.ShapeDtypeStruct((_KP,), _f32),
        jax.ShapeDtypeStruct((_KP,), _f32),
        jax.ShapeDtypeStruct((_KP,), _f32),
        jax.ShapeDtypeStruct((_KP,), _f32),
        jax.ShapeDtypeStruct((_KP,), _i32),
    ),
    mesh=_mesh,
    compiler_params=pltpu.CompilerParams(needs_layout_passes=False),
    scratch_types=[
        pltpu.VMEM((_CHUNK,), _f32),
        pltpu.VMEM((_CHUNK,), _f32),
        pltpu.VMEM((_CHUNK,), _f32),
        pltpu.VMEM((_CHUNK,), _f32),
        pltpu.VMEM((_CHUNK,), _f32),
        pltpu.VMEM((_NGT,), _f32),
        pltpu.VMEM((_NGT,), _f32),
        pltpu.VMEM((_NGT,), _f32),
        pltpu.VMEM((_NGT,), _f32),
        pltpu.VMEM((_NGT,), _i32),
        pltpu.VMEM((_CHUNK,), _i32),
        pltpu.VMEM((_CHUNK,), _i32),
        pltpu.VMEM((16,), _f32),
        pltpu.VMEM((_NS * 16,), _f32),
        pltpu.VMEM((_KP,), _f32),
        pltpu.VMEM((_KP,), _f32),
        pltpu.VMEM((_KP,), _f32),
        pltpu.VMEM((_KP,), _f32),
        pltpu.VMEM((_KP,), _f32),
        pltpu.VMEM((_KP,), _i32),
        pltpu.VMEM((_KP,), _i32),
        pltpu.VMEM_SHARED((_NS * 16,), _f32),
        pltpu.SMEM((1,), _i32),
    ],
)


@jax.jit
def kernel(boxes, scores, gt_boxes, gt_classes):
    bt = jnp.pad(boxes.T, ((0, 0), (0, _NP - _N)))
    psc = jnp.pad(scores, (0, _NP - _N))
    gt = gt_boxes.T
    mi, pc, ki, ks, kx1, ky1, kx2, ky2, kcls = _roiheads_sc(
        bt[0], bt[1], bt[2], bt[3], psc,
        gt[0], gt[1], gt[2], gt[3], gt_classes.astype(jnp.int32))
    kept_boxes = jnp.stack([kx1[:_K], ky1[:_K], kx2[:_K], ky2[:_K]], axis=1)
    return (kept_boxes, ks[:_K], kcls[:_K], pc[:_N], mi[:_N])


# speedup vs baseline: 9.4272x; 1.4123x over previous
import functools
import jax
import jax.numpy as jnp
from jax import lax
from jax.experimental import pallas as pl
from jax.experimental.pallas import tpu as pltpu
from jax.experimental.pallas import tpu_sc as plsc

_NUM_CLASSES = 80
_SCORE_THRESH = 0.05
_NMS_THRESH = 0.5
_IOU_FG_THRESH = 0.5
_N = 5000
_NP = 5120
_NGT = 64
_K = 100
_KP = 128
_NS = 16
_CHUNK = _NP // _NS
_NB = _CHUNK // 16
_BIG = 0x7FFFFFFF


def _body(px1, py1, px2, py2, psc, gx1, gy1, gx2, gy2, gcls,
          mi_out, pc_out, ki_out, ks_out, kx1_out, ky1_out, kx2_out,
          ky2_out, kcls_out,
          x1v, y1v, x2v, y2v, scv, gx1v, gy1v, gx2v, gy2v, gclsv,
          miv, pcv, rowv, candv, kx1v, ky1v, kx2v, ky2v, ksv, kiv,
          kclsv, cand_sh):
    sid = lax.axis_index("s")
    base = sid * _CHUNK
    iota = lax.iota(jnp.int32, 16)
    lane0 = iota == 0

    rowv[...] = jnp.zeros((16,), jnp.float32)
    pltpu.sync_copy(rowv, cand_sh.at[pl.ds(sid * 16, 16)])
    pltpu.sync_copy(rowv, cand_sh.at[pl.ds(256 + sid * 16, 16)])

    pltpu.sync_copy(px1.at[pl.ds(base, _CHUNK)], x1v)
    pltpu.sync_copy(py1.at[pl.ds(base, _CHUNK)], y1v)
    pltpu.sync_copy(px2.at[pl.ds(base, _CHUNK)], x2v)
    pltpu.sync_copy(py2.at[pl.ds(base, _CHUNK)], y2v)
    pltpu.sync_copy(psc.at[pl.ds(base, _CHUNK)], scv)
    pltpu.sync_copy(gx1, gx1v)
    pltpu.sync_copy(gy1, gy1v)
    pltpu.sync_copy(gx2, gx2v)
    pltpu.sync_copy(gy2, gy2v)
    pltpu.sync_copy(gcls, gclsv)

    bval0 = jnp.full((16,), -jnp.inf, jnp.float32)
    bblk0 = jnp.zeros((16,), jnp.int32)
    for b in range(_NB):
        off = b * 16
        q1 = x1v[pl.ds(off, 16)]
        q2 = y1v[pl.ds(off, 16)]
        q3 = x2v[pl.ds(off, 16)]
        q4 = y2v[pl.ds(off, 16)]
        parea = (q3 - q1) * (q4 - q2)

        def gt_body(j, carry):
            biou, bj = carry
            jv = jnp.full((16,), j, jnp.int32)
            g1 = plsc.load_gather(gx1v, [jv])
            g2 = plsc.load_gather(gy1v, [jv])
            g3 = plsc.load_gather(gx2v, [jv])
            g4 = plsc.load_gather(gy2v, [jv])
            garea = (g3 - g1) * (g4 - g2)
            ww = jnp.maximum(jnp.minimum(g3, q3) - jnp.maximum(g1, q1), 0.0)
            hh = jnp.maximum(jnp.minimum(g4, q4) - jnp.maximum(g2, q2), 0.0)
            inter = ww * hh
            iou = inter / jnp.maximum(garea + parea - inter, 1e-8)
            upd = iou > biou
            return jnp.where(upd, iou, biou), jnp.where(upd, jv, bj)

        biou, bj = lax.fori_loop(
            0, _NGT, gt_body,
            (jnp.full((16,), -1.0, jnp.float32), jnp.zeros((16,), jnp.int32)))
        cls_g = plsc.load_gather(gclsv, [bj])
        pcls = jnp.where(biou >= _IOU_FG_THRESH, cls_g,
                         jnp.full((16,), _NUM_CLASSES, jnp.int32))
        miv[pl.ds(off, 16)] = bj
        pcv[pl.ds(off, 16)] = pcls
        s = scv[pl.ds(off, 16)]
        s = jnp.where(s > _SCORE_THRESH, s, -1.0)
        gidx = jnp.full((16,), base + off, jnp.int32) + iota
        s = jnp.where(gidx < _N, s, -2.0)
        scv[pl.ds(off, 16)] = s
        upd0 = s > bval0
        bval0 = jnp.where(upd0, s, bval0)
        bblk0 = jnp.where(upd0, jnp.full((16,), b, jnp.int32), bblk0)

    pltpu.sync_copy(miv, mi_out.at[pl.ds(base, _CHUNK)])
    pltpu.sync_copy(pcv, pc_out.at[pl.ds(base, _CHUNK)])
    plsc.subcore_barrier()

    def nms_body(t, carry):
        bval, bblk = carry
        pbase = (t & 1) * 256
        m = jnp.max(bval)
        gi = jnp.full((16,), base, jnp.int32) + bblk * 16 + iota
        li = jnp.min(jnp.where(bval == m, gi, _BIG))
        locv = jnp.full((16,), li - base, jnp.int32)
        c1 = plsc.load_gather(x1v, [locv])
        c2 = plsc.load_gather(y1v, [locv])
        c3 = plsc.load_gather(x2v, [locv])
        c4 = plsc.load_gather(y2v, [locv])
        c5 = plsc.load_gather(pcv, [locv]).astype(jnp.float32)
        lif = jnp.full((16,), li, jnp.int32).astype(jnp.float32)
        mf = jnp.full((16,), m, jnp.float32)
        tagv = jnp.full((16,), t + 1, jnp.int32).astype(jnp.float32)
        row = jnp.where(iota == 0, mf,
              jnp.where(iota == 1, lif,
              jnp.where(iota == 2, c1,
              jnp.where(iota == 3, c2,
              jnp.where(iota == 4, c3,
              jnp.where(iota == 5, c4,
              jnp.where(iota == 6, c5, tagv)))))))
        rowv[...] = row
        pltpu.sync_copy(rowv, cand_sh.at[pl.ds(pbase + sid * 16, 16)])
        plsc.subcore_barrier()

        pltpu.sync_copy(cand_sh.at[pl.ds(pbase, 256)], candv)
        rbase = iota * 16
        for _ in range(2):
            tags = plsc.load_gather(candv, [rbase + 7])
            stale = jnp.logical_not(jnp.all(tags == tagv))

            @pl.when(stale)
            def _():
                pltpu.sync_copy(cand_sh.at[pl.ds(pbase, 256)], candv)

        sv = plsc.load_gather(candv, [rbase])
        ivf = plsc.load_gather(candv, [rbase + 1])
        iv = ivf.astype(jnp.int32)
        gm = jnp.max(sv)
        winmask = sv == gm
        gbi = jnp.min(jnp.where(winmask, iv, _BIG))
        wlane = jnp.min(jnp.where(winmask & (iv == gbi), iota,
                                  jnp.full((16,), 99, jnp.int32)))
        wb = jnp.full((16,), wlane * 16, jnp.int32)
        bx1 = plsc.load_gather(candv, [wb + 2])
        by1 = plsc.load_gather(candv, [wb + 3])
        bx2 = plsc.load_gather(candv, [wb + 4])
        by2 = plsc.load_gather(candv, [wb + 5])
        bcl = plsc.load_gather(candv, [wb + 6])
        tv = jnp.full((16,), t, jnp.int32)
        plsc.store_scatter(kiv, [tv], jnp.full((16,), gbi, jnp.int32),
                           mask=lane0)
        plsc.store_scatter(ksv, [tv], jnp.full((16,), gm, jnp.float32),
                           mask=lane0)
        plsc.store_scatter(kx1v, [tv], bx1, mask=lane0)
        plsc.store_scatter(ky1v, [tv], by1, mask=lane0)
        plsc.store_scatter(kx2v, [tv], bx2, mask=lane0)
        plsc.store_scatter(ky2v, [tv], by2, mask=lane0)
        plsc.store_scatter(kclsv, [tv], bcl.astype(jnp.int32), mask=lane0)
        barea = (bx2 - bx1) * (by2 - by1)
        nval = jnp.full((16,), -jnp.inf, jnp.float32)
        nblk = jnp.zeros((16,), jnp.int32)
        for b in range(_NB):
            off = b * 16
            q1 = x1v[pl.ds(off, 16)]
            q2 = y1v[pl.ds(off, 16)]
            q3 = x2v[pl.ds(off, 16)]
            q4 = y2v[pl.ds(off, 16)]
            ww = jnp.maximum(jnp.minimum(bx2, q3) - jnp.maximum(bx1, q1), 0.0)
            hh = jnp.maximum(jnp.minimum(by2, q4) - jnp.maximum(by1, q2), 0.0)
            inter = ww * hh
            qarea = (q3 - q1) * (q4 - q2)
            iou = inter / jnp.maximum(barea + qarea - inter, 1e-8)
            s = scv[pl.ds(off, 16)]
            s = jnp.where(iou > _NMS_THRESH, -1.0, s)
            scv[pl.ds(off, 16)] = s
            nupd = s > nval
            nval = jnp.where(nupd, s, nval)
            nblk = jnp.where(nupd, jnp.full((16,), b, jnp.int32), nblk)
        return nval, nblk

    lax.fori_loop(0, _K, nms_body, (bval0, bblk0))

    @pl.when(sid == 0)
    def _():
        pltpu.sync_copy(kiv, ki_out)
        pltpu.sync_copy(ksv, ks_out)
        pltpu.sync_copy(kx1v, kx1_out)
        pltpu.sync_copy(ky1v, ky1_out)
        pltpu.sync_copy(kx2v, kx2_out)
        pltpu.sync_copy(ky2v, ky2_out)
        pltpu.sync_copy(kclsv, kcls_out)


_mesh = plsc.VectorSubcoreMesh(core_axis_name="c", subcore_axis_name="s",
                               num_cores=1)

_f32 = jnp.float32
_i32 = jnp.int32

_roiheads_sc = pl.kernel(
    _body,
    out_type=(
        jax.ShapeDtypeStruct((_NP,), _i32),
        jax.ShapeDtypeStruct((_NP,), _i32),
        jax.ShapeDtypeStruct((_KP,), _i32),
        jax.ShapeDtypeStruct((_KP,), _f32),
        jax.ShapeDtypeStruct((_KP,), _f32),
        jax.ShapeDtypeStruct((_KP,), _f32),
        jax.ShapeDtypeStruct((_KP,), _f32),
        jax.ShapeDtypeStruct((_KP,), _f32),
        jax.ShapeDtypeStruct((_KP,), _i32),
    ),
    mesh=_mesh,
    compiler_params=pltpu.CompilerParams(needs_layout_passes=False),
    scratch_types=[
        pltpu.VMEM((_CHUNK,), _f32),
        pltpu.VMEM((_CHUNK,), _f32),
        pltpu.VMEM((_CHUNK,), _f32),
        pltpu.VMEM((_CHUNK,), _f32),
        pltpu.VMEM((_CHUNK,), _f32),
        pltpu.VMEM((_NGT,), _f32),
        pltpu.VMEM((_NGT,), _f32),
        pltpu.VMEM((_NGT,), _f32),
        pltpu.VMEM((_NGT,), _f32),
        pltpu.VMEM((_NGT,), _i32),
        pltpu.VMEM((_CHUNK,), _i32),
        pltpu.VMEM((_CHUNK,), _i32),
        pltpu.VMEM((16,), _f32),
        pltpu.VMEM((_NS * 16,), _f32),
        pltpu.VMEM((_KP,), _f32),
        pltpu.VMEM((_KP,), _f32),
        pltpu.VMEM((_KP,), _f32),
        pltpu.VMEM((_KP,), _f32),
        pltpu.VMEM((_KP,), _f32),
        pltpu.VMEM((_KP,), _i32),
        pltpu.VMEM((_KP,), _i32),
        pltpu.VMEM_SHARED((2 * _NS * 16,), _f32),
    ],
)


@jax.jit
def kernel(boxes, scores, gt_boxes, gt_classes):
    bt = jnp.pad(boxes.T, ((0, 0), (0, _NP - _N)))
    psc = jnp.pad(scores, (0, _NP - _N))
    gt = gt_boxes.T
    mi, pc, ki, ks, kx1, ky1, kx2, ky2, kcls = _roiheads_sc(
        bt[0], bt[1], bt[2], bt[3], psc,
        gt[0], gt[1], gt[2], gt[3], gt_classes.astype(jnp.int32))
    kept_boxes = jnp.stack([kx1[:_K], ky1[:_K], kx2[:_K], ky2[:_K]], axis=1)
    return (kept_boxes, ks[:_K], kcls[:_K], pc[:_N], mi[:_N])


# speedup vs baseline: 9.7990x; 1.0394x over previous
import functools
import jax
import jax.numpy as jnp
from jax import lax
from jax.experimental import pallas as pl
from jax.experimental.pallas import tpu as pltpu
from jax.experimental.pallas import tpu_sc as plsc

_NUM_CLASSES = 80
_SCORE_THRESH = 0.05
_NMS_THRESH = 0.5
_IOU_FG_THRESH = 0.5
_N = 5000
_NP = 5120
_NGT = 64
_K = 100
_KP = 128
_NS = 16
_CHUNK = _NP // _NS
_NB = _CHUNK // 16
_BIG = 0x7FFFFFFF


def _body(px1, py1, px2, py2, psc, gx1, gy1, gx2, gy2, gcls,
          mi_out, pc_out, ki_out, ks_out, kx1_out, ky1_out, kx2_out,
          ky2_out, kcls_out,
          x1v, y1v, x2v, y2v, scv, gx1v, gy1v, gx2v, gy2v, gclsv,
          miv, pcv, areav, rowv, candv, kx1v, ky1v, kx2v, ky2v, ksv, kiv,
          kclsv, cand_sh):
    sid = lax.axis_index("s")
    base = sid * _CHUNK
    iota = lax.iota(jnp.int32, 16)
    lane0 = iota == 0

    rowv[...] = jnp.zeros((16,), jnp.float32)
    pltpu.sync_copy(rowv, cand_sh.at[pl.ds(sid * 16, 16)])
    pltpu.sync_copy(rowv, cand_sh.at[pl.ds(256 + sid * 16, 16)])

    pltpu.sync_copy(px1.at[pl.ds(base, _CHUNK)], x1v)
    pltpu.sync_copy(py1.at[pl.ds(base, _CHUNK)], y1v)
    pltpu.sync_copy(px2.at[pl.ds(base, _CHUNK)], x2v)
    pltpu.sync_copy(py2.at[pl.ds(base, _CHUNK)], y2v)
    pltpu.sync_copy(psc.at[pl.ds(base, _CHUNK)], scv)
    pltpu.sync_copy(gx1, gx1v)
    pltpu.sync_copy(gy1, gy1v)
    pltpu.sync_copy(gx2, gx2v)
    pltpu.sync_copy(gy2, gy2v)
    pltpu.sync_copy(gcls, gclsv)

    bval0 = jnp.full((16,), -jnp.inf, jnp.float32)
    bblk0 = jnp.zeros((16,), jnp.int32)
    for b in range(_NB):
        off = b * 16
        q1 = x1v[pl.ds(off, 16)]
        q2 = y1v[pl.ds(off, 16)]
        q3 = x2v[pl.ds(off, 16)]
        q4 = y2v[pl.ds(off, 16)]
        parea = (q3 - q1) * (q4 - q2)

        def gt_body(j, carry):
            biou, bj = carry
            jv = jnp.full((16,), j, jnp.int32)
            g1 = plsc.load_gather(gx1v, [jv])
            g2 = plsc.load_gather(gy1v, [jv])
            g3 = plsc.load_gather(gx2v, [jv])
            g4 = plsc.load_gather(gy2v, [jv])
            garea = (g3 - g1) * (g4 - g2)
            ww = jnp.maximum(jnp.minimum(g3, q3) - jnp.maximum(g1, q1), 0.0)
            hh = jnp.maximum(jnp.minimum(g4, q4) - jnp.maximum(g2, q2), 0.0)
            inter = ww * hh
            iou = inter / jnp.maximum(garea + parea - inter, 1e-8)
            upd = iou > biou
            return jnp.where(upd, iou, biou), jnp.where(upd, jv, bj)

        biou, bj = lax.fori_loop(
            0, _NGT, gt_body,
            (jnp.full((16,), -1.0, jnp.float32), jnp.zeros((16,), jnp.int32)))
        cls_g = plsc.load_gather(gclsv, [bj])
        pcls = jnp.where(biou >= _IOU_FG_THRESH, cls_g,
                         jnp.full((16,), _NUM_CLASSES, jnp.int32))
        miv[pl.ds(off, 16)] = bj
        pcv[pl.ds(off, 16)] = pcls
        areav[pl.ds(off, 16)] = parea
        s = scv[pl.ds(off, 16)]
        s = jnp.where(s > _SCORE_THRESH, s, -1.0)
        gidx = jnp.full((16,), base + off, jnp.int32) + iota
        s = jnp.where(gidx < _N, s, -2.0)
        scv[pl.ds(off, 16)] = s
        upd0 = s > bval0
        bval0 = jnp.where(upd0, s, bval0)
        bblk0 = jnp.where(upd0, jnp.full((16,), b, jnp.int32), bblk0)

    pltpu.sync_copy(miv, mi_out.at[pl.ds(base, _CHUNK)])
    pltpu.sync_copy(pcv, pc_out.at[pl.ds(base, _CHUNK)])
    plsc.subcore_barrier()

    def nms_body(t, carry):
        bval, bblk = carry
        pbase = (t & 1) * 256
        m = jnp.max(bval)
        gi = jnp.full((16,), base, jnp.int32) + bblk * 16 + iota
        li = jnp.min(jnp.where(bval == m, gi, _BIG))
        locv = jnp.full((16,), li - base, jnp.int32)
        c1 = plsc.load_gather(x1v, [locv])
        c2 = plsc.load_gather(y1v, [locv])
        c3 = plsc.load_gather(x2v, [locv])
        c4 = plsc.load_gather(y2v, [locv])
        c5 = plsc.load_gather(pcv, [locv]).astype(jnp.float32)
        lif = jnp.full((16,), li, jnp.int32).astype(jnp.float32)
        mf = jnp.full((16,), m, jnp.float32)
        tagv = jnp.full((16,), t + 1, jnp.int32).astype(jnp.float32)
        row = jnp.where(iota == 0, mf,
              jnp.where(iota == 1, lif,
              jnp.where(iota == 2, c1,
              jnp.where(iota == 3, c2,
              jnp.where(iota == 4, c3,
              jnp.where(iota == 5, c4,
              jnp.where(iota == 6, c5, tagv)))))))
        rowv[...] = row
        pltpu.sync_copy(rowv, cand_sh.at[pl.ds(pbase + sid * 16, 16)])
        plsc.subcore_barrier()

        pltpu.sync_copy(cand_sh.at[pl.ds(pbase, 256)], candv)
        rbase = iota * 16
        for _ in range(2):
            tags = plsc.load_gather(candv, [rbase + 7])
            stale = jnp.logical_not(jnp.all(tags == tagv))

            @pl.when(stale)
            def _():
                pltpu.sync_copy(cand_sh.at[pl.ds(pbase, 256)], candv)

        sv = plsc.load_gather(candv, [rbase])
        ivf = plsc.load_gather(candv, [rbase + 1])
        iv = ivf.astype(jnp.int32)
        gm = jnp.max(sv)
        winmask = sv == gm
        kmin = jnp.min(jnp.where(winmask, iv * 16 + iota, _BIG))
        gbi = kmin >> 4
        wb = jnp.full((16,), (kmin & 15) * 16, jnp.int32)
        bx1 = plsc.load_gather(candv, [wb + 2])
        by1 = plsc.load_gather(candv, [wb + 3])
        bx2 = plsc.load_gather(candv, [wb + 4])
        by2 = plsc.load_gather(candv, [wb + 5])
        bcl = plsc.load_gather(candv, [wb + 6])
        tv = jnp.full((16,), t, jnp.int32)
        plsc.store_scatter(kiv, [tv], jnp.full((16,), gbi, jnp.int32),
                           mask=lane0)
        plsc.store_scatter(ksv, [tv], jnp.full((16,), gm, jnp.float32),
                           mask=lane0)
        plsc.store_scatter(kx1v, [tv], bx1, mask=lane0)
        plsc.store_scatter(ky1v, [tv], by1, mask=lane0)
        plsc.store_scatter(kx2v, [tv], bx2, mask=lane0)
        plsc.store_scatter(ky2v, [tv], by2, mask=lane0)
        plsc.store_scatter(kclsv, [tv], bcl.astype(jnp.int32), mask=lane0)
        barea = (bx2 - bx1) * (by2 - by1)
        nval = jnp.full((16,), -jnp.inf, jnp.float32)
        nblk = jnp.zeros((16,), jnp.int32)
        for b in range(_NB):
            off = b * 16
            q1 = x1v[pl.ds(off, 16)]
            q2 = y1v[pl.ds(off, 16)]
            q3 = x2v[pl.ds(off, 16)]
            q4 = y2v[pl.ds(off, 16)]
            ww = jnp.maximum(jnp.minimum(bx2, q3) - jnp.maximum(bx1, q1), 0.0)
            hh = jnp.maximum(jnp.minimum(by2, q4) - jnp.maximum(by1, q2), 0.0)
            inter = ww * hh
            qarea = areav[pl.ds(off, 16)]
            iou = inter / jnp.maximum(barea + qarea - inter, 1e-8)
            s = scv[pl.ds(off, 16)]
            s = jnp.where(iou > _NMS_THRESH, -1.0, s)
            scv[pl.ds(off, 16)] = s
            nupd = s > nval
            nval = jnp.where(nupd, s, nval)
            nblk = jnp.where(nupd, jnp.full((16,), b, jnp.int32), nblk)
        return nval, nblk

    lax.fori_loop(0, _K, nms_body, (bval0, bblk0))

    @pl.when(sid == 0)
    def _():
        pltpu.sync_copy(kiv, ki_out)
        pltpu.sync_copy(ksv, ks_out)
        pltpu.sync_copy(kx1v, kx1_out)
        pltpu.sync_copy(ky1v, ky1_out)
        pltpu.sync_copy(kx2v, kx2_out)
        pltpu.sync_copy(ky2v, ky2_out)
        pltpu.sync_copy(kclsv, kcls_out)


_mesh = plsc.VectorSubcoreMesh(core_axis_name="c", subcore_axis_name="s",
                               num_cores=1)

_f32 = jnp.float32
_i32 = jnp.int32

_roiheads_sc = pl.kernel(
    _body,
    out_type=(
        jax.ShapeDtypeStruct((_NP,), _i32),
        jax.ShapeDtypeStruct((_NP,), _i32),
        jax.ShapeDtypeStruct((_KP,), _i32),
        jax.ShapeDtypeStruct((_KP,), _f32),
        jax.ShapeDtypeStruct((_KP,), _f32),
        jax.ShapeDtypeStruct((_KP,), _f32),
        jax.ShapeDtypeStruct((_KP,), _f32),
        jax.ShapeDtypeStruct((_KP,), _f32),
        jax.ShapeDtypeStruct((_KP,), _i32),
    ),
    mesh=_mesh,
    compiler_params=pltpu.CompilerParams(needs_layout_passes=False),
    scratch_types=[
        pltpu.VMEM((_CHUNK,), _f32),
        pltpu.VMEM((_CHUNK,), _f32),
        pltpu.VMEM((_CHUNK,), _f32),
        pltpu.VMEM((_CHUNK,), _f32),
        pltpu.VMEM((_CHUNK,), _f32),
        pltpu.VMEM((_NGT,), _f32),
        pltpu.VMEM((_NGT,), _f32),
        pltpu.VMEM((_NGT,), _f32),
        pltpu.VMEM((_NGT,), _f32),
        pltpu.VMEM((_NGT,), _i32),
        pltpu.VMEM((_CHUNK,), _i32),
        pltpu.VMEM((_CHUNK,), _i32),
        pltpu.VMEM((_CHUNK,), _f32),
        pltpu.VMEM((16,), _f32),
        pltpu.VMEM((_NS * 16,), _f32),
        pltpu.VMEM((_KP,), _f32),
        pltpu.VMEM((_KP,), _f32),
        pltpu.VMEM((_KP,), _f32),
        pltpu.VMEM((_KP,), _f32),
        pltpu.VMEM((_KP,), _f32),
        pltpu.VMEM((_KP,), _i32),
        pltpu.VMEM((_KP,), _i32),
        pltpu.VMEM_SHARED((2 * _NS * 16,), _f32),
    ],
)


@jax.jit
def kernel(boxes, scores, gt_boxes, gt_classes):
    bt = jnp.pad(boxes.T, ((0, 0), (0, _NP - _N)))
    psc = jnp.pad(scores, (0, _NP - _N))
    gt = gt_boxes.T
    mi, pc, ki, ks, kx1, ky1, kx2, ky2, kcls = _roiheads_sc(
        bt[0], bt[1], bt[2], bt[3], psc,
        gt[0], gt[1], gt[2], gt[3], gt_classes.astype(jnp.int32))
    kept_boxes = jnp.stack([kx1[:_K], ky1[:_K], kx2[:_K], ky2[:_K]], axis=1)
    return (kept_boxes, ks[:_K], kcls[:_K], pc[:_N], mi[:_N])


# speedup vs baseline: 9.8042x; 1.0005x over previous
import jax
import jax.numpy as jnp
from jax import lax
from jax.experimental import pallas as pl
from jax.experimental.pallas import tpu as pltpu
from jax.experimental.pallas import tpu_sc as plsc

_NUM_CLASSES = 80
_SCORE_THRESH = 0.05
_NMS_THRESH = 0.5
_IOU_FG_THRESH = 0.5
_N = 5000
_NP = 5120
_NGT = 64
_K = 100
_KP = 128
_NS = 16
_CHUNK = _NP // _NS
_NB = _CHUNK // 16
_BIG = 0x7FFFFFFF


def _body(px1, py1, px2, py2, psc, gx1, gy1, gx2, gy2, gcls,
          mi_out, pc_out, ki_out, ks_out, kx1_out, ky1_out, kx2_out,
          ky2_out, kcls_out,
          x1v, y1v, x2v, y2v, scv, gx1v, gy1v, gx2v, gy2v, gclsv,
          miv, pcv, areav, rowv, candv, kx1v, ky1v, kx2v, ky2v, ksv, kiv,
          kclsv, cand_sh):
    sid = lax.axis_index("s")
    base = sid * _CHUNK
    iota = lax.iota(jnp.int32, 16)
    lane0 = iota == 0

    rowv[...] = jnp.zeros((16,), jnp.float32)
    pltpu.sync_copy(rowv, cand_sh.at[pl.ds(sid * 16, 16)])
    pltpu.sync_copy(rowv, cand_sh.at[pl.ds(256 + sid * 16, 16)])

    pltpu.sync_copy(px1.at[pl.ds(base, _CHUNK)], x1v)
    pltpu.sync_copy(py1.at[pl.ds(base, _CHUNK)], y1v)
    pltpu.sync_copy(px2.at[pl.ds(base, _CHUNK)], x2v)
    pltpu.sync_copy(py2.at[pl.ds(base, _CHUNK)], y2v)
    pltpu.sync_copy(psc.at[pl.ds(base, _CHUNK)], scv)
    pltpu.sync_copy(gx1, gx1v)
    pltpu.sync_copy(gy1, gy1v)
    pltpu.sync_copy(gx2, gx2v)
    pltpu.sync_copy(gy2, gy2v)
    pltpu.sync_copy(gcls, gclsv)

    bval0 = jnp.full((16,), -jnp.inf, jnp.float32)
    bblk0 = jnp.zeros((16,), jnp.int32)
    for b in range(_NB):
        off = b * 16
        q1 = x1v[pl.ds(off, 16)]
        q2 = y1v[pl.ds(off, 16)]
        q3 = x2v[pl.ds(off, 16)]
        q4 = y2v[pl.ds(off, 16)]
        parea = (q3 - q1) * (q4 - q2)

        def gt_body(j, carry):
            biou, bj = carry
            jv = jnp.full((16,), j, jnp.int32)
            g1 = plsc.load_gather(gx1v, [jv])
            g2 = plsc.load_gather(gy1v, [jv])
            g3 = plsc.load_gather(gx2v, [jv])
            g4 = plsc.load_gather(gy2v, [jv])
            garea = (g3 - g1) * (g4 - g2)
            ww = jnp.maximum(jnp.minimum(g3, q3) - jnp.maximum(g1, q1), 0.0)
            hh = jnp.maximum(jnp.minimum(g4, q4) - jnp.maximum(g2, q2), 0.0)
            inter = ww * hh
            iou = inter / jnp.maximum(garea + parea - inter, 1e-8)
            upd = iou > biou
            return jnp.where(upd, iou, biou), jnp.where(upd, jv, bj)

        biou, bj = lax.fori_loop(
            0, _NGT, gt_body,
            (jnp.full((16,), -1.0, jnp.float32), jnp.zeros((16,), jnp.int32)))
        cls_g = plsc.load_gather(gclsv, [bj])
        pcls = jnp.where(biou >= _IOU_FG_THRESH, cls_g,
                         jnp.full((16,), _NUM_CLASSES, jnp.int32))
        miv[pl.ds(off, 16)] = bj
        pcv[pl.ds(off, 16)] = pcls
        areav[pl.ds(off, 16)] = parea
        s = scv[pl.ds(off, 16)]
        s = jnp.where(s > _SCORE_THRESH, s, -1.0)
        gidx = jnp.full((16,), base + off, jnp.int32) + iota
        s = jnp.where(gidx < _N, s, -2.0)
        scv[pl.ds(off, 16)] = s
        upd0 = s > bval0
        bval0 = jnp.where(upd0, s, bval0)
        bblk0 = jnp.where(upd0, jnp.full((16,), b, jnp.int32), bblk0)

    pltpu.sync_copy(miv, mi_out.at[pl.ds(base, _CHUNK)])
    pltpu.sync_copy(pcv, pc_out.at[pl.ds(base, _CHUNK)])
    plsc.subcore_barrier()

    def nms_body(t, carry):
        bval, bblk = carry
        pbase = (t & 1) * 256
        m = jnp.max(bval)
        gi = jnp.full((16,), base, jnp.int32) + bblk * 16 + iota
        li = jnp.min(jnp.where(bval == m, gi, _BIG))
        locv = jnp.full((16,), li - base, jnp.int32)
        c1 = plsc.load_gather(x1v, [locv])
        c2 = plsc.load_gather(y1v, [locv])
        c3 = plsc.load_gather(x2v, [locv])
        c4 = plsc.load_gather(y2v, [locv])
        c5 = plsc.load_gather(pcv, [locv]).astype(jnp.float32)
        lif = jnp.full((16,), li, jnp.int32).astype(jnp.float32)
        mf = jnp.full((16,), m, jnp.float32)
        tagv = jnp.full((16,), t + 1, jnp.int32).astype(jnp.float32)
        row = jnp.where(iota == 0, mf,
              jnp.where(iota == 1, lif,
              jnp.where(iota == 2, c1,
              jnp.where(iota == 3, c2,
              jnp.where(iota == 4, c3,
              jnp.where(iota == 5, c4,
              jnp.where(iota == 6, c5, tagv)))))))
        rowv[...] = row
        pltpu.sync_copy(rowv, cand_sh.at[pl.ds(pbase + sid * 16, 16)])
        plsc.subcore_barrier()

        pltpu.sync_copy(cand_sh.at[pl.ds(pbase, 256)], candv)
        rbase = iota * 16
        for _ in range(2):
            tags = plsc.load_gather(candv, [rbase + 7])
            stale = jnp.logical_not(jnp.all(tags == tagv))

            @pl.when(stale)
            def _():
                pltpu.sync_copy(cand_sh.at[pl.ds(pbase, 256)], candv)

        sv = plsc.load_gather(candv, [rbase])
        ivf = plsc.load_gather(candv, [rbase + 1])
        iv = ivf.astype(jnp.int32)
        gm = jnp.max(sv)
        winmask = sv == gm
        kmin = jnp.min(jnp.where(winmask, iv * 16 + iota, _BIG))
        gbi = kmin >> 4
        wb = jnp.full((16,), (kmin & 15) * 16, jnp.int32)
        bx1 = plsc.load_gather(candv, [wb + 2])
        by1 = plsc.load_gather(candv, [wb + 3])
        bx2 = plsc.load_gather(candv, [wb + 4])
        by2 = plsc.load_gather(candv, [wb + 5])
        bcl = plsc.load_gather(candv, [wb + 6])
        tv = jnp.full((16,), t, jnp.int32)
        plsc.store_scatter(kiv, [tv], jnp.full((16,), gbi, jnp.int32),
                           mask=lane0)
        plsc.store_scatter(ksv, [tv], jnp.full((16,), gm, jnp.float32),
                           mask=lane0)
        plsc.store_scatter(kx1v, [tv], bx1, mask=lane0)
        plsc.store_scatter(ky1v, [tv], by1, mask=lane0)
        plsc.store_scatter(kx2v, [tv], bx2, mask=lane0)
        plsc.store_scatter(ky2v, [tv], by2, mask=lane0)
        plsc.store_scatter(kclsv, [tv], bcl.astype(jnp.int32), mask=lane0)
        barea = (bx2 - bx1) * (by2 - by1)
        nval = jnp.full((16,), -jnp.inf, jnp.float32)
        nblk = jnp.zeros((16,), jnp.int32)
        for b in range(_NB):
            off = b * 16
            q1 = x1v[pl.ds(off, 16)]
            q2 = y1v[pl.ds(off, 16)]
            q3 = x2v[pl.ds(off, 16)]
            q4 = y2v[pl.ds(off, 16)]
            ww = jnp.maximum(jnp.minimum(bx2, q3) - jnp.maximum(bx1, q1), 0.0)
            hh = jnp.maximum(jnp.minimum(by2, q4) - jnp.maximum(by1, q2), 0.0)
            inter = ww * hh
            qarea = areav[pl.ds(off, 16)]
            iou = inter / jnp.maximum(barea + qarea - inter, 1e-8)
            s = scv[pl.ds(off, 16)]
            s = jnp.where(iou > _NMS_THRESH, -1.0, s)
            scv[pl.ds(off, 16)] = s
            nupd = s > nval
            nval = jnp.where(nupd, s, nval)
            nblk = jnp.where(nupd, jnp.full((16,), b, jnp.int32), nblk)
        return nval, nblk

    lax.fori_loop(0, _K, nms_body, (bval0, bblk0))

    @pl.when(sid == 0)
    def _():
        pltpu.sync_copy(kiv, ki_out)
        pltpu.sync_copy(ksv, ks_out)
        pltpu.sync_copy(kx1v, kx1_out)
        pltpu.sync_copy(ky1v, ky1_out)
        pltpu.sync_copy(kx2v, kx2_out)
        pltpu.sync_copy(ky2v, ky2_out)
        pltpu.sync_copy(kclsv, kcls_out)


_mesh = plsc.VectorSubcoreMesh(core_axis_name="c", subcore_axis_name="s",
                               num_cores=1)

_f32 = jnp.float32
_i32 = jnp.int32

_roiheads_sc = pl.kernel(
    _body,
    out_type=(
        jax.ShapeDtypeStruct((_NP,), _i32),
        jax.ShapeDtypeStruct((_NP,), _i32),
        jax.ShapeDtypeStruct((_KP,), _i32),
        jax.ShapeDtypeStruct((_KP,), _f32),
        jax.ShapeDtypeStruct((_KP,), _f32),
        jax.ShapeDtypeStruct((_KP,), _f32),
        jax.ShapeDtypeStruct((_KP,), _f32),
        jax.ShapeDtypeStruct((_KP,), _f32),
        jax.ShapeDtypeStruct((_KP,), _i32),
    ),
    mesh=_mesh,
    compiler_params=pltpu.CompilerParams(needs_layout_passes=False),
    scratch_types=[
        pltpu.VMEM((_CHUNK,), _f32),
        pltpu.VMEM((_CHUNK,), _f32),
        pltpu.VMEM((_CHUNK,), _f32),
        pltpu.VMEM((_CHUNK,), _f32),
        pltpu.VMEM((_CHUNK,), _f32),
        pltpu.VMEM((_NGT,), _f32),
        pltpu.VMEM((_NGT,), _f32),
        pltpu.VMEM((_NGT,), _f32),
        pltpu.VMEM((_NGT,), _f32),
        pltpu.VMEM((_NGT,), _i32),
        pltpu.VMEM((_CHUNK,), _i32),
        pltpu.VMEM((_CHUNK,), _i32),
        pltpu.VMEM((_CHUNK,), _f32),
        pltpu.VMEM((16,), _f32),
        pltpu.VMEM((_NS * 16,), _f32),
        pltpu.VMEM((_KP,), _f32),
        pltpu.VMEM((_KP,), _f32),
        pltpu.VMEM((_KP,), _f32),
        pltpu.VMEM((_KP,), _f32),
        pltpu.VMEM((_KP,), _f32),
        pltpu.VMEM((_KP,), _i32),
        pltpu.VMEM((_KP,), _i32),
        pltpu.VMEM_SHARED((2 * _NS * 16,), _f32),
    ],
)


@jax.jit
def kernel(boxes, scores, gt_boxes, gt_classes):
    bt = jnp.pad(boxes.T, ((0, 0), (0, _NP - _N)))
    psc = jnp.pad(scores, (0, _NP - _N))
    gt = gt_boxes.T
    mi, pc, ki, ks, kx1, ky1, kx2, ky2, kcls = _roiheads_sc(
        bt[0], bt[1], bt[2], bt[3], psc,
        gt[0], gt[1], gt[2], gt[3], gt_classes.astype(jnp.int32))
    kept_boxes = jnp.stack([kx1[:_K], ky1[:_K], kx2[:_K], ky2[:_K]], axis=1)
    return (kept_boxes, ks[:_K], kcls[:_K], pc[:_N], mi[:_N])
